# baseline (speedup 1.0000x reference)
import functools

import jax
import jax.numpy as jnp
from jax import lax
from jax.experimental import pallas as pl
from jax.experimental.pallas import tpu as pltpu
from jax.experimental.pallas import tpu_sc as plsc

B = 8
CIN = 3
H = 512
W = 512
P = 16
C = 96
N = 1024
K = 9
D_FF = 4 * C
EMB = 256
K16 = 16
CP = 128

NUM_WORKERS = 32
NODES_PER_WORKER = (B * N) // NUM_WORKERS
CHUNK_NODES = 8
NUM_CHUNKS = NODES_PER_WORKER // CHUNK_NODES


def _gelu(x):
    return jax.nn.gelu(x)


TROWS = 128
SROWS = 64


def _dist_topk_store(hsrc_ref, idx_ref, d_scr, ht_scr, x2_scr, b):
    base = b * N
    col = lax.broadcasted_iota(jnp.int32, (SROWS, N), 1)
    klane = lax.broadcasted_iota(jnp.int32, (SROWS, K16), 1)

    def t_body(i, _):
        blk = hsrc_ref[0, pl.ds(i * TROWS, TROWS), 0:C]
        ht_scr[:, pl.ds(i * TROWS, TROWS)] = blk.T
        return 0

    lax.fori_loop(0, N // TROWS, t_body, 0)

    htf = ht_scr[...]
    x2_scr[...] = jnp.broadcast_to(jnp.sum(htf * htf, axis=0)[None, :], (8, N))

    def tile_body(i, _):
        ht = hsrc_ref[0, pl.ds(i * TROWS, TROWS), 0:C]
        hh = jnp.dot(ht, ht_scr[...], preferred_element_type=jnp.float32)
        d_scr[...] = x2_scr[0:1, :] - 2.0 * hh

        def sub_body(s, _):
            row0 = i * TROWS + s * SROWS
            d = d_scr[pl.ds(s * SROWS, SROWS), :]
            rid = lax.broadcasted_iota(jnp.int32, (SROWS, K16), 0)
            acc = rid + (base + row0)
            for k in range(K):
                amin = jnp.argmin(d, axis=1).astype(jnp.int32)
                d = jnp.where(col == amin[:, None], jnp.float32(jnp.inf), d)
                acc = jnp.where(klane == k, (amin + base)[:, None], acc)
            idx_ref[0, pl.ds(row0, SROWS), :] = acc
            return 0

        lax.fori_loop(0, TROWS // SROWS, sub_body, 0)
        return 0

    lax.fori_loop(0, N // TROWS, tile_body, 0)


def _pad_cp(h):
    return jnp.concatenate([h, jnp.zeros((N, CP - C), jnp.float32)], axis=1)


def _stem_kernel(p_ref, w_ref, b_ref, pos_ref, h_ref, idx_ref, d_scr, ht_scr, x2_scr):
    b = pl.program_id(0)
    h = jnp.dot(p_ref[0], w_ref[...], preferred_element_type=jnp.float32)
    h = h + b_ref[...] + pos_ref[0]
    h_ref[0] = _pad_cp(h)
    _dist_topk_store(h_ref, idx_ref, d_scr, ht_scr, x2_scr, b)


def _grapher_ffn(h, rmax, wg_ref, bg_ref, wa_ref, ba_ref, wb_ref, bb_ref):
    rel = rmax - h
    g = jnp.dot(h, wg_ref[:C], preferred_element_type=jnp.float32)
    g = g + jnp.dot(rel, wg_ref[C:], preferred_element_type=jnp.float32)
    g = _gelu(g + bg_ref[...])
    x1 = h + g
    f = _gelu(jnp.dot(x1, wa_ref[...], preferred_element_type=jnp.float32)
              + ba_ref[...])
    f = jnp.dot(f, wb_ref[...], preferred_element_type=jnp.float32) + bb_ref[...]
    return x1 + f


def _block_topk_kernel(h_ref, rmax_ref, wg_ref, bg_ref, wa_ref, ba_ref,
                       wb_ref, bb_ref, hout_ref, idx_ref, d_scr, ht_scr,
                       x2_scr):
    b = pl.program_id(0)
    hn = _grapher_ffn(h_ref[0][:, :C], rmax_ref[0][:, :C], wg_ref, bg_ref,
                      wa_ref, ba_ref, wb_ref, bb_ref)
    hout_ref[0] = _pad_cp(hn)
    _dist_topk_store(hout_ref, idx_ref, d_scr, ht_scr, x2_scr, b)


def _final_kernel(h_ref, rmax_ref, wg_ref, bg_ref, wa_ref, ba_ref, wb_ref,
                  bb_ref, pw_ref, pb_ref, out_ref):
    hn = _grapher_ffn(h_ref[0][:, :C], rmax_ref[0][:, :C], wg_ref, bg_ref,
                      wa_ref, ba_ref, wb_ref, bb_ref)
    feat = jnp.mean(hn, axis=0)[None, :]
    out_ref[0] = (jnp.dot(feat, pw_ref[...], preferred_element_type=jnp.float32)
                  + pb_ref[...])


def _sc_gather_max_body(h_hbm, idx_hbm, out_hbm, idx_v, rows_v, out_v, sem):
    wid = lax.axis_index("s") * 2 + lax.axis_index("c")
    nbase = wid * NODES_PER_WORKER
    pltpu.sync_copy(idx_hbm.at[pl.ds(nbase * K16, NODES_PER_WORKER * K16)],
                    idx_v)

    def chunk_body(i, carry):
        off = pl.multiple_of(i * (CHUNK_NODES * K16), 8)
        pltpu.async_copy(
            h_hbm.at[idx_v.at[pl.ds(off, CHUNK_NODES * K16)]], rows_v, sem
        ).wait()
        for j in range(CHUNK_NODES):
            for c in range(CP // 16):
                m = rows_v[j * K16, pl.ds(c * 16, 16)]
                for k in range(1, K):
                    m = jnp.maximum(m, rows_v[j * K16 + k, pl.ds(c * 16, 16)])
                out_v[j, pl.ds(c * 16, 16)] = m
        pltpu.sync_copy(out_v,
                        out_hbm.at[pl.ds(nbase + i * CHUNK_NODES, CHUNK_NODES)])
        return carry

    lax.fori_loop(0, NUM_CHUNKS, chunk_body, 0)


def _make_sc_gather_max():
    mesh = plsc.VectorSubcoreMesh(core_axis_name="c", subcore_axis_name="s")
    return pl.kernel(
        _sc_gather_max_body,
        mesh=mesh,
        compiler_params=pltpu.CompilerParams(use_tc_tiling_on_sc=True),
        out_type=jax.ShapeDtypeStruct((B * N, CP), jnp.float32),
        scratch_types=[
            pltpu.VMEM((NODES_PER_WORKER * K16,), jnp.int32),
            pltpu.VMEM((CHUNK_NODES * K16, CP), jnp.float32),
            pltpu.VMEM((CHUNK_NODES, CP), jnp.float32),
            pltpu.SemaphoreType.DMA,
        ],
    )


def kernel(x, stem_w, stem_b, pos_embed, Wg1, bg1, Wf1a, bf1a, Wf1b, bf1b,
           Wg2, bg2, Wf2a, bf2a, Wf2b, bf2b, proj_w, proj_b):
    p = x.reshape(B, CIN, H // P, P, W // P, P)
    p = p.transpose(0, 2, 4, 1, 3, 5).reshape(B, N, CIN * P * P)

    stem_b = stem_b.reshape(1, C)
    bg1 = bg1.reshape(1, C)
    bg2 = bg2.reshape(1, C)
    bf1a = bf1a.reshape(1, D_FF)
    bf2a = bf2a.reshape(1, D_FF)
    bf1b = bf1b.reshape(1, C)
    bf2b = bf2b.reshape(1, C)
    proj_b = proj_b.reshape(1, EMB)

    full = lambda shape: pl.BlockSpec(shape, lambda b: (0,) * len(shape))
    per_img = lambda shape: pl.BlockSpec((1,) + shape, lambda b: (b, 0, 0))

    h0, idx1 = pl.pallas_call(
        _stem_kernel,
        grid=(B,),
        in_specs=[
            per_img((N, CIN * P * P)),
            full((CIN * P * P, C)),
            full((1, C)),
            full((1, N, C)),
        ],
        out_specs=[per_img((N, CP)), per_img((N, K16))],
        out_shape=[
            jax.ShapeDtypeStruct((B, N, CP), jnp.float32),
            jax.ShapeDtypeStruct((B, N, K16), jnp.int32),
        ],
        scratch_shapes=[pltpu.VMEM((TROWS, N), jnp.float32),
                        pltpu.VMEM((C, N), jnp.float32),
                        pltpu.VMEM((8, N), jnp.float32)],
    )(p, stem_w, stem_b, pos_embed)

    sc_gather_max = _make_sc_gather_max()

    rmax1 = sc_gather_max(h0.reshape(B * N, CP), idx1.reshape(B * N * K16))

    h1, idx2 = pl.pallas_call(
        _block_topk_kernel,
        grid=(B,),
        in_specs=[
            per_img((N, CP)),
            per_img((N, CP)),
            full((2 * C, C)),
            full((1, C)),
            full((C, D_FF)),
            full((1, D_FF)),
            full((D_FF, C)),
            full((1, C)),
        ],
        out_specs=[per_img((N, CP)), per_img((N, K16))],
        out_shape=[
            jax.ShapeDtypeStruct((B, N, CP), jnp.float32),
            jax.ShapeDtypeStruct((B, N, K16), jnp.int32),
        ],
        scratch_shapes=[pltpu.VMEM((TROWS, N), jnp.float32),
                        pltpu.VMEM((C, N), jnp.float32),
                        pltpu.VMEM((8, N), jnp.float32)],
    )(h0, rmax1.reshape(B, N, CP), Wg1, bg1, Wf1a, bf1a, Wf1b, bf1b)

    rmax2 = sc_gather_max(h1.reshape(B * N, CP), idx2.reshape(B * N * K16))

    out = pl.pallas_call(
        _final_kernel,
        grid=(B,),
        in_specs=[
            per_img((N, CP)),
            per_img((N, CP)),
            full((2 * C, C)),
            full((1, C)),
            full((C, D_FF)),
            full((1, D_FF)),
            full((D_FF, C)),
            full((1, C)),
            full((C, EMB)),
            full((1, EMB)),
        ],
        out_specs=pl.BlockSpec((1, 1, EMB), lambda b: (b, 0, 0)),
        out_shape=jax.ShapeDtypeStruct((B, 1, EMB), jnp.float32),
    )(h1, rmax2.reshape(B, N, CP), Wg2, bg2, Wf2a, bf2a, Wf2b, bf2b,
      proj_w, proj_b)

    return out.reshape(B, EMB)


# speedup vs baseline: 1.3341x; 1.3341x over previous
import functools

import jax
import jax.numpy as jnp
from jax import lax
from jax.experimental import pallas as pl
from jax.experimental.pallas import tpu as pltpu
from jax.experimental.pallas import tpu_sc as plsc

B = 8
CIN = 3
H = 512
W = 512
P = 16
C = 96
N = 1024
K = 9
D_FF = 4 * C
EMB = 256
CP = 128

NUM_WORKERS = 32
NODES_PER_WORKER = (B * N) // NUM_WORKERS
CHUNK_NODES = 8
NUM_CHUNKS = NODES_PER_WORKER // CHUNK_NODES


def _gelu(x):
    return jax.nn.gelu(x)


TROWS = 128
SROWS = 128


def _dist_topk_store(hsrc_ref, idx_ref, d_scr, ht_scr, x2_scr, b):
    base = b * N
    col = lax.broadcasted_iota(jnp.int32, (SROWS, N), 1)
    klane = lax.broadcasted_iota(jnp.int32, (SROWS, K), 1)

    def t_body(i, _):
        blk = hsrc_ref[0, pl.ds(i * TROWS, TROWS), 0:C]
        ht_scr[:, pl.ds(i * TROWS, TROWS)] = blk.T
        return 0

    lax.fori_loop(0, N // TROWS, t_body, 0)

    htf = ht_scr[...]
    x2_scr[...] = jnp.broadcast_to(jnp.sum(htf * htf, axis=0)[None, :], (8, N))

    def tile_body(i, _):
        ht = hsrc_ref[0, pl.ds(i * TROWS, TROWS), 0:C]
        hh = jnp.dot(ht, ht_scr[...], preferred_element_type=jnp.float32)
        d_scr[...] = x2_scr[0:1, :] - 2.0 * hh

        def sub_body(s, _):
            d = d_scr[pl.ds(s * SROWS, SROWS), :]
            acc = jnp.zeros((SROWS, K), jnp.int32)
            for k in range(K):
                amin = jnp.argmin(d, axis=1).astype(jnp.int32)
                d = jnp.where(col == amin[:, None], jnp.float32(jnp.inf), d)
                acc = jnp.where(klane == k, (amin + base)[:, None], acc)
            idx_ref[0, pl.ds(i * TROWS + s * SROWS, SROWS), :] = acc
            return 0

        lax.fori_loop(0, TROWS // SROWS, sub_body, 0)
        return 0

    lax.fori_loop(0, N // TROWS, tile_body, 0)


def _pad_cp(h):
    return jnp.concatenate([h, jnp.zeros((N, CP - C), jnp.float32)], axis=1)


def _stem_kernel(p_ref, w_ref, b_ref, pos_ref, h_ref, idx_ref, d_scr, ht_scr, x2_scr):
    b = pl.program_id(0)
    h = jnp.dot(p_ref[0], w_ref[...], preferred_element_type=jnp.float32)
    h = h + b_ref[...] + pos_ref[0]
    h_ref[0] = _pad_cp(h)
    _dist_topk_store(h_ref, idx_ref, d_scr, ht_scr, x2_scr, b)


def _grapher_ffn(h, rmax, wg_ref, bg_ref, wa_ref, ba_ref, wb_ref, bb_ref):
    rel = rmax - h
    g = jnp.dot(h, wg_ref[:C], preferred_element_type=jnp.float32)
    g = g + jnp.dot(rel, wg_ref[C:], preferred_element_type=jnp.float32)
    g = _gelu(g + bg_ref[...])
    x1 = h + g
    f = _gelu(jnp.dot(x1, wa_ref[...], preferred_element_type=jnp.float32)
              + ba_ref[...])
    f = jnp.dot(f, wb_ref[...], preferred_element_type=jnp.float32) + bb_ref[...]
    return x1 + f


def _block_topk_kernel(h_ref, rmax_ref, wg_ref, bg_ref, wa_ref, ba_ref,
                       wb_ref, bb_ref, hout_ref, idx_ref, d_scr, ht_scr,
                       x2_scr):
    b = pl.program_id(0)
    hn = _grapher_ffn(h_ref[0][:, :C], rmax_ref[0][:, :C], wg_ref, bg_ref,
                      wa_ref, ba_ref, wb_ref, bb_ref)
    hout_ref[0] = _pad_cp(hn)
    _dist_topk_store(hout_ref, idx_ref, d_scr, ht_scr, x2_scr, b)


def _final_kernel(h_ref, rmax_ref, wg_ref, bg_ref, wa_ref, ba_ref, wb_ref,
                  bb_ref, pw_ref, pb_ref, out_ref):
    hn = _grapher_ffn(h_ref[0][:, :C], rmax_ref[0][:, :C], wg_ref, bg_ref,
                      wa_ref, ba_ref, wb_ref, bb_ref)
    feat = jnp.mean(hn, axis=0)[None, :]
    out_ref[0] = (jnp.dot(feat, pw_ref[...], preferred_element_type=jnp.float32)
                  + pb_ref[...])


def _sc_gather_max_body(h_hbm, idx_hbm, out_hbm, idx_v, rows_v, out_v, sem):
    wid = lax.axis_index("s") * 2 + lax.axis_index("c")
    nbase = wid * NODES_PER_WORKER
    pltpu.sync_copy(idx_hbm.at[pl.ds(nbase * K, NODES_PER_WORKER * K)], idx_v)

    def chunk_body(i, carry):
        off = pl.multiple_of(i * (CHUNK_NODES * K), 8)
        pltpu.async_copy(
            h_hbm.at[idx_v.at[pl.ds(off, CHUNK_NODES * K)]], rows_v, sem
        ).wait()
        for j in range(CHUNK_NODES):
            for c in range(CP // 16):
                m = rows_v[j * K, pl.ds(c * 16, 16)]
                for k in range(1, K):
                    m = jnp.maximum(m, rows_v[j * K + k, pl.ds(c * 16, 16)])
                out_v[j, pl.ds(c * 16, 16)] = m
        pltpu.sync_copy(out_v,
                        out_hbm.at[pl.ds(nbase + i * CHUNK_NODES, CHUNK_NODES)])
        return carry

    lax.fori_loop(0, NUM_CHUNKS, chunk_body, 0)


def _make_sc_gather_max():
    mesh = plsc.VectorSubcoreMesh(core_axis_name="c", subcore_axis_name="s")
    return pl.kernel(
        _sc_gather_max_body,
        mesh=mesh,
        compiler_params=pltpu.CompilerParams(use_tc_tiling_on_sc=True),
        out_type=jax.ShapeDtypeStruct((B * N, CP), jnp.float32),
        scratch_types=[
            pltpu.VMEM((NODES_PER_WORKER * K,), jnp.int32),
            pltpu.VMEM((CHUNK_NODES * K, CP), jnp.float32),
            pltpu.VMEM((CHUNK_NODES, CP), jnp.float32),
            pltpu.SemaphoreType.DMA,
        ],
    )


def kernel(x, stem_w, stem_b, pos_embed, Wg1, bg1, Wf1a, bf1a, Wf1b, bf1b,
           Wg2, bg2, Wf2a, bf2a, Wf2b, bf2b, proj_w, proj_b):
    p = x.reshape(B, CIN, H // P, P, W // P, P)
    p = p.transpose(0, 2, 4, 1, 3, 5).reshape(B, N, CIN * P * P)

    stem_b = stem_b.reshape(1, C)
    bg1 = bg1.reshape(1, C)
    bg2 = bg2.reshape(1, C)
    bf1a = bf1a.reshape(1, D_FF)
    bf2a = bf2a.reshape(1, D_FF)
    bf1b = bf1b.reshape(1, C)
    bf2b = bf2b.reshape(1, C)
    proj_b = proj_b.reshape(1, EMB)

    full = lambda shape: pl.BlockSpec(shape, lambda b: (0,) * len(shape))
    per_img = lambda shape: pl.BlockSpec((1,) + shape, lambda b: (b, 0, 0))

    h0, idx1 = pl.pallas_call(
        _stem_kernel,
        grid=(B,),
        in_specs=[
            per_img((N, CIN * P * P)),
            full((CIN * P * P, C)),
            full((1, C)),
            full((1, N, C)),
        ],
        out_specs=[per_img((N, CP)), per_img((N, K))],
        out_shape=[
            jax.ShapeDtypeStruct((B, N, CP), jnp.float32),
            jax.ShapeDtypeStruct((B, N, K), jnp.int32),
        ],
        scratch_shapes=[pltpu.VMEM((TROWS, N), jnp.float32),
                        pltpu.VMEM((C, N), jnp.float32),
                        pltpu.VMEM((8, N), jnp.float32)],
    )(p, stem_w, stem_b, pos_embed)

    sc_gather_max = _make_sc_gather_max()

    rmax1 = sc_gather_max(h0.reshape(B * N, CP), idx1.reshape(B * N * K))

    h1, idx2 = pl.pallas_call(
        _block_topk_kernel,
        grid=(B,),
        in_specs=[
            per_img((N, CP)),
            per_img((N, CP)),
            full((2 * C, C)),
            full((1, C)),
            full((C, D_FF)),
            full((1, D_FF)),
            full((D_FF, C)),
            full((1, C)),
        ],
        out_specs=[per_img((N, CP)), per_img((N, K))],
        out_shape=[
            jax.ShapeDtypeStruct((B, N, CP), jnp.float32),
            jax.ShapeDtypeStruct((B, N, K), jnp.int32),
        ],
        scratch_shapes=[pltpu.VMEM((TROWS, N), jnp.float32),
                        pltpu.VMEM((C, N), jnp.float32),
                        pltpu.VMEM((8, N), jnp.float32)],
    )(h0, rmax1.reshape(B, N, CP), Wg1, bg1, Wf1a, bf1a, Wf1b, bf1b)

    rmax2 = sc_gather_max(h1.reshape(B * N, CP), idx2.reshape(B * N * K))

    out = pl.pallas_call(
        _final_kernel,
        grid=(B,),
        in_specs=[
            per_img((N, CP)),
            per_img((N, CP)),
            full((2 * C, C)),
            full((1, C)),
            full((C, D_FF)),
            full((1, D_FF)),
            full((D_FF, C)),
            full((1, C)),
            full((C, EMB)),
            full((1, EMB)),
        ],
        out_specs=pl.BlockSpec((1, 1, EMB), lambda b: (b, 0, 0)),
        out_shape=jax.ShapeDtypeStruct((B, 1, EMB), jnp.float32),
    )(h1, rmax2.reshape(B, N, CP), Wg2, bg2, Wf2a, bf2a, Wf2b, bf2b,
      proj_w, proj_b)

    return out.reshape(B, EMB)


# speedup vs baseline: 1.5462x; 1.1589x over previous
import functools

import jax
import jax.numpy as jnp
from jax import lax
from jax.experimental import pallas as pl
from jax.experimental.pallas import tpu as pltpu
from jax.experimental.pallas import tpu_sc as plsc

B = 8
CIN = 3
H = 512
W = 512
P = 16
C = 96
N = 1024
K = 9
D_FF = 4 * C
EMB = 256
CP = 128

NUM_WORKERS = 32
NODES_PER_WORKER = (B * N) // NUM_WORKERS
CHUNK_NODES = 8
NUM_CHUNKS = NODES_PER_WORKER // CHUNK_NODES


def _gelu(x):
    return jax.nn.gelu(x)


TROWS = 256
SROWS = 256


def _dist_topk_store(hsrc_ref, idx_ref, d_scr, ht_scr, x2_scr, b):
    base = b * N
    col = lax.broadcasted_iota(jnp.int32, (SROWS, N), 1)
    klane = lax.broadcasted_iota(jnp.int32, (SROWS, K), 1)

    def t_body(i, _):
        blk = hsrc_ref[0, pl.ds(i * TROWS, TROWS), 0:C]
        ht_scr[:, pl.ds(i * TROWS, TROWS)] = blk.T
        return 0

    lax.fori_loop(0, N // TROWS, t_body, 0)

    htf = ht_scr[...]
    x2_scr[...] = jnp.broadcast_to(jnp.sum(htf * htf, axis=0)[None, :], (8, N))

    def tile_body(i, _):
        ht = hsrc_ref[0, pl.ds(i * TROWS, TROWS), 0:C]
        hh = jnp.dot(ht, ht_scr[...], preferred_element_type=jnp.float32)
        d_scr[...] = x2_scr[0:1, :] - 2.0 * hh

        def sub_body(s, _):
            d = d_scr[pl.ds(s * SROWS, SROWS), :]
            acc = jnp.zeros((SROWS, K), jnp.int32)
            for k in range(K):
                amin = jnp.argmin(d, axis=1).astype(jnp.int32)
                d = jnp.where(col == amin[:, None], jnp.float32(jnp.inf), d)
                acc = jnp.where(klane == k, (amin + base)[:, None], acc)
            idx_ref[0, pl.ds(i * TROWS + s * SROWS, SROWS), :] = acc
            return 0

        lax.fori_loop(0, TROWS // SROWS, sub_body, 0)
        return 0

    lax.fori_loop(0, N // TROWS, tile_body, 0)


def _pad_cp(h):
    return jnp.concatenate([h, jnp.zeros((N, CP - C), jnp.float32)], axis=1)


def _stem_kernel(p_ref, w_ref, b_ref, pos_ref, h_ref, idx_ref, d_scr, ht_scr, x2_scr):
    b = pl.program_id(0)
    h = jnp.dot(p_ref[0], w_ref[...], preferred_element_type=jnp.float32)
    h = h + b_ref[...] + pos_ref[0]
    h_ref[0] = _pad_cp(h)
    _dist_topk_store(h_ref, idx_ref, d_scr, ht_scr, x2_scr, b)


def _grapher_ffn(h, rmax, wg_ref, bg_ref, wa_ref, ba_ref, wb_ref, bb_ref):
    rel = rmax - h
    g = jnp.dot(h, wg_ref[:C], preferred_element_type=jnp.float32)
    g = g + jnp.dot(rel, wg_ref[C:], preferred_element_type=jnp.float32)
    g = _gelu(g + bg_ref[...])
    x1 = h + g
    f = _gelu(jnp.dot(x1, wa_ref[...], preferred_element_type=jnp.float32)
              + ba_ref[...])
    f = jnp.dot(f, wb_ref[...], preferred_element_type=jnp.float32) + bb_ref[...]
    return x1 + f


def _block_topk_kernel(h_ref, rmax_ref, wg_ref, bg_ref, wa_ref, ba_ref,
                       wb_ref, bb_ref, hout_ref, idx_ref, d_scr, ht_scr,
                       x2_scr):
    b = pl.program_id(0)
    hn = _grapher_ffn(h_ref[0][:, :C], rmax_ref[0][:, :C], wg_ref, bg_ref,
                      wa_ref, ba_ref, wb_ref, bb_ref)
    hout_ref[0] = _pad_cp(hn)
    _dist_topk_store(hout_ref, idx_ref, d_scr, ht_scr, x2_scr, b)


def _final_kernel(h_ref, rmax_ref, wg_ref, bg_ref, wa_ref, ba_ref, wb_ref,
                  bb_ref, pw_ref, pb_ref, out_ref):
    hn = _grapher_ffn(h_ref[0][:, :C], rmax_ref[0][:, :C], wg_ref, bg_ref,
                      wa_ref, ba_ref, wb_ref, bb_ref)
    feat = jnp.mean(hn, axis=0)[None, :]
    out_ref[0] = (jnp.dot(feat, pw_ref[...], preferred_element_type=jnp.float32)
                  + pb_ref[...])


def _sc_gather_max_body(h_hbm, idx_hbm, out_hbm, idx_v, rows_v, out_v, sem):
    wid = lax.axis_index("s") * 2 + lax.axis_index("c")
    nbase = wid * NODES_PER_WORKER
    pltpu.sync_copy(idx_hbm.at[pl.ds(nbase * K, NODES_PER_WORKER * K)], idx_v)

    def chunk_body(i, carry):
        off = pl.multiple_of(i * (CHUNK_NODES * K), 8)
        pltpu.async_copy(
            h_hbm.at[idx_v.at[pl.ds(off, CHUNK_NODES * K)]], rows_v, sem
        ).wait()
        for j in range(CHUNK_NODES):
            for c in range(CP // 16):
                m = rows_v[j * K, pl.ds(c * 16, 16)]
                for k in range(1, K):
                    m = jnp.maximum(m, rows_v[j * K + k, pl.ds(c * 16, 16)])
                out_v[j, pl.ds(c * 16, 16)] = m
        pltpu.sync_copy(out_v,
                        out_hbm.at[pl.ds(nbase + i * CHUNK_NODES, CHUNK_NODES)])
        return carry

    lax.fori_loop(0, NUM_CHUNKS, chunk_body, 0)


def _make_sc_gather_max():
    mesh = plsc.VectorSubcoreMesh(core_axis_name="c", subcore_axis_name="s")
    return pl.kernel(
        _sc_gather_max_body,
        mesh=mesh,
        compiler_params=pltpu.CompilerParams(use_tc_tiling_on_sc=True),
        out_type=jax.ShapeDtypeStruct((B * N, CP), jnp.float32),
        scratch_types=[
            pltpu.VMEM((NODES_PER_WORKER * K,), jnp.int32),
            pltpu.VMEM((CHUNK_NODES * K, CP), jnp.float32),
            pltpu.VMEM((CHUNK_NODES, CP), jnp.float32),
            pltpu.SemaphoreType.DMA,
        ],
    )


def kernel(x, stem_w, stem_b, pos_embed, Wg1, bg1, Wf1a, bf1a, Wf1b, bf1b,
           Wg2, bg2, Wf2a, bf2a, Wf2b, bf2b, proj_w, proj_b):
    p = x.reshape(B, CIN, H // P, P, W // P, P)
    p = p.transpose(0, 2, 4, 1, 3, 5).reshape(B, N, CIN * P * P)

    stem_b = stem_b.reshape(1, C)
    bg1 = bg1.reshape(1, C)
    bg2 = bg2.reshape(1, C)
    bf1a = bf1a.reshape(1, D_FF)
    bf2a = bf2a.reshape(1, D_FF)
    bf1b = bf1b.reshape(1, C)
    bf2b = bf2b.reshape(1, C)
    proj_b = proj_b.reshape(1, EMB)

    full = lambda shape: pl.BlockSpec(shape, lambda b: (0,) * len(shape))
    per_img = lambda shape: pl.BlockSpec((1,) + shape, lambda b: (b, 0, 0))

    h0, idx1 = pl.pallas_call(
        _stem_kernel,
        grid=(B,),
        in_specs=[
            per_img((N, CIN * P * P)),
            full((CIN * P * P, C)),
            full((1, C)),
            full((1, N, C)),
        ],
        out_specs=[per_img((N, CP)), per_img((N, K))],
        out_shape=[
            jax.ShapeDtypeStruct((B, N, CP), jnp.float32),
            jax.ShapeDtypeStruct((B, N, K), jnp.int32),
        ],
        scratch_shapes=[pltpu.VMEM((TROWS, N), jnp.float32),
                        pltpu.VMEM((C, N), jnp.float32),
                        pltpu.VMEM((8, N), jnp.float32)],
    )(p, stem_w, stem_b, pos_embed)

    sc_gather_max = _make_sc_gather_max()

    rmax1 = sc_gather_max(h0.reshape(B * N, CP), idx1.reshape(B * N * K))

    h1, idx2 = pl.pallas_call(
        _block_topk_kernel,
        grid=(B,),
        in_specs=[
            per_img((N, CP)),
            per_img((N, CP)),
            full((2 * C, C)),
            full((1, C)),
            full((C, D_FF)),
            full((1, D_FF)),
            full((D_FF, C)),
            full((1, C)),
        ],
        out_specs=[per_img((N, CP)), per_img((N, K))],
        out_shape=[
            jax.ShapeDtypeStruct((B, N, CP), jnp.float32),
            jax.ShapeDtypeStruct((B, N, K), jnp.int32),
        ],
        scratch_shapes=[pltpu.VMEM((TROWS, N), jnp.float32),
                        pltpu.VMEM((C, N), jnp.float32),
                        pltpu.VMEM((8, N), jnp.float32)],
    )(h0, rmax1.reshape(B, N, CP), Wg1, bg1, Wf1a, bf1a, Wf1b, bf1b)

    rmax2 = sc_gather_max(h1.reshape(B * N, CP), idx2.reshape(B * N * K))

    out = pl.pallas_call(
        _final_kernel,
        grid=(B,),
        in_specs=[
            per_img((N, CP)),
            per_img((N, CP)),
            full((2 * C, C)),
            full((1, C)),
            full((C, D_FF)),
            full((1, D_FF)),
            full((D_FF, C)),
            full((1, C)),
            full((C, EMB)),
            full((1, EMB)),
        ],
        out_specs=pl.BlockSpec((1, 1, EMB), lambda b: (b, 0, 0)),
        out_shape=jax.ShapeDtypeStruct((B, 1, EMB), jnp.float32),
    )(h1, rmax2.reshape(B, N, CP), Wg2, bg2, Wf2a, bf2a, Wf2b, bf2b,
      proj_w, proj_b)

    return out.reshape(B, EMB)
